# Initial kernel scaffold; baseline (speedup 1.0000x reference)
import jax
import jax.numpy as jnp
from jax.experimental import pallas as pl


def kernel(letter_idx, letter_embed):
    raise NotImplementedError("write your pallas kernel here")


# speedup vs baseline: 12.8705x; 12.8705x over previous
import jax
import jax.numpy as jnp
from jax import lax
from jax.experimental import pallas as pl
from jax.experimental.pallas import tpu as pltpu
from jax.experimental.pallas import tpu_sc as plsc

_B = 16384
_T = 200
_D = 16
_V = 26
_NC = 2
_NS = 16
_NW = _NC * _NS
_BW = _B // _NW
_TS = 4
_STEPS = _T // _TS
_L = 16
_G = _BW // _L


def _body(table_hbm, idxt_hbm, out_hbm, table_v, idx_v, out_v, sem):
    wid = lax.axis_index("s") * _NC + lax.axis_index("c")
    b0 = wid * _BW

    pltpu.sync_copy(table_hbm, table_v)

    def step(s, carry):
        t0 = s * _TS
        pltpu.sync_copy(idxt_hbm.at[pl.ds(t0, _TS), pl.ds(b0, _BW)], idx_v)

        def group(g, carry2):
            c0 = g * _L
            for r in range(_TS):
                v = idx_v[r, pl.ds(c0, _L)]
                va = v * _D
                for d in range(_D):
                    out_v[r * _D + d, pl.ds(c0, _L)] = plsc.load_gather(
                        table_v, [va + d]
                    )
            return carry2

        lax.fori_loop(0, _G, group, 0)
        pltpu.sync_copy(out_v, out_hbm.at[pl.ds(t0 * _D, _TS * _D), pl.ds(b0, _BW)])
        return carry

    lax.fori_loop(0, _STEPS, step, 0)


def kernel(letter_idx, letter_embed):
    idxt = letter_idx.astype(jnp.int32).T
    table = letter_embed.reshape(_V * _D).astype(jnp.float32)

    mesh = plsc.VectorSubcoreMesh(core_axis_name="c", subcore_axis_name="s")
    k = pl.kernel(
        _body,
        mesh=mesh,
        compiler_params=pltpu.CompilerParams(needs_layout_passes=False),
        out_type=jax.ShapeDtypeStruct((_T * _D, _B), jnp.float32),
        scratch_types=[
            pltpu.VMEM((_V * _D,), jnp.float32),
            pltpu.VMEM((_TS, _BW), jnp.int32),
            pltpu.VMEM((_TS * _D, _BW), jnp.float32),
            pltpu.SemaphoreType.DMA,
        ],
    )
    out2 = k(table, idxt)
    return out2.reshape(_T, _D, _B).transpose(2, 0, 1)


# speedup vs baseline: 14.3369x; 1.1139x over previous
import jax
import jax.numpy as jnp
from jax import lax
from jax.experimental import pallas as pl
from jax.experimental.pallas import tpu as pltpu
from jax.experimental.pallas import tpu_sc as plsc

_B = 16384
_T = 200
_D = 16
_V = 26
_NC = 2
_NS = 16
_NW = _NC * _NS
_BW = _B // _NW
_TS = 4
_STEPS = _T // _TS
_L = 16
_G = _BW // _L


def _body(table_hbm, idxt_hbm, out_hbm, table_v,
          idx_v0, idx_v1, out_v0, out_v1,
          isem0, isem1, osem0, osem1, tsem):
    wid = lax.axis_index("s") * _NC + lax.axis_index("c")
    b0 = wid * _BW
    idx_bufs = (idx_v0, idx_v1)
    out_bufs = (out_v0, out_v1)
    isems = (isem0, isem1)
    osems = (osem0, osem1)

    pltpu.async_copy(table_hbm, table_v, tsem).wait()

    def idx_src(s):
        return idxt_hbm.at[pl.ds(s * _TS, _TS), pl.ds(b0, _BW)]

    def out_dst(s):
        return out_hbm.at[pl.ds(s * _TS * _D, _TS * _D), pl.ds(b0, _BW)]

    pltpu.async_copy(idx_src(0), idx_v0, isem0)
    pltpu.async_copy(idx_src(1), idx_v1, isem1)

    def outer(i, carry):
        for p in range(2):
            s = i * 2 + p
            idx_v = idx_bufs[p]
            out_v = out_bufs[p]
            pltpu.make_async_copy(idx_src(s), idx_v, isems[p]).wait()

            @pl.when(i > 0)
            def _():
                pltpu.make_async_copy(out_v, out_dst(s - 2), osems[p]).wait()

            def group(g, carry2):
                c0 = g * _L
                for r in range(_TS):
                    v = idx_v[r, pl.ds(c0, _L)]
                    va = v * _D
                    for d in range(_D):
                        out_v[r * _D + d, pl.ds(c0, _L)] = plsc.load_gather(
                            table_v, [va + d]
                        )
                return carry2

            lax.fori_loop(0, _G, group, 0)
            pltpu.async_copy(out_v, out_dst(s), osems[p])

            @pl.when(i < (_STEPS // 2 - 1))
            def _():
                pltpu.async_copy(idx_src(s + 2), idx_v, isems[p])

        return carry

    lax.fori_loop(0, _STEPS // 2, outer, 0)
    pltpu.make_async_copy(out_v0, out_dst(_STEPS - 2), osem0).wait()
    pltpu.make_async_copy(out_v1, out_dst(_STEPS - 1), osem1).wait()


def kernel(letter_idx, letter_embed):
    idxt = letter_idx.astype(jnp.int32).T
    table = letter_embed.reshape(_V * _D).astype(jnp.float32)

    mesh = plsc.VectorSubcoreMesh(core_axis_name="c", subcore_axis_name="s")
    k = pl.kernel(
        _body,
        mesh=mesh,
        compiler_params=pltpu.CompilerParams(needs_layout_passes=False),
        out_type=jax.ShapeDtypeStruct((_T * _D, _B), jnp.float32),
        scratch_types=[
            pltpu.VMEM((_V * _D,), jnp.float32),
            pltpu.VMEM((_TS, _BW), jnp.int32),
            pltpu.VMEM((_TS, _BW), jnp.int32),
            pltpu.VMEM((_TS * _D, _BW), jnp.float32),
            pltpu.VMEM((_TS * _D, _BW), jnp.float32),
            pltpu.SemaphoreType.DMA,
            pltpu.SemaphoreType.DMA,
            pltpu.SemaphoreType.DMA,
            pltpu.SemaphoreType.DMA,
            pltpu.SemaphoreType.DMA,
        ],
    )
    out2 = k(table, idxt)
    return out2.reshape(_T, _D, _B).transpose(2, 0, 1)


# speedup vs baseline: 38.3758x; 2.6767x over previous
import jax
import jax.numpy as jnp
from jax import lax
from jax.experimental import pallas as pl
from jax.experimental.pallas import tpu as pltpu
from jax.experimental.pallas import tpu_sc as plsc

_B = 16384
_T = 200
_D = 16
_V = 26
_NC = 2
_NS = 16
_NW = _NC * _NS
_BW = _B // _NW
_TS = 4
_STEPS = _T // _TS
_L = 16
_G = _BW // _L


def _body(table_hbm, idxt_hbm, out_hbm, table_v,
          idx_v0, idx_v1, out_v0, out_v1,
          isem0, isem1, osem0, osem1, tsem):
    wid = lax.axis_index("s") * _NC + lax.axis_index("c")
    b0 = wid * _BW
    idx_bufs = (idx_v0, idx_v1)
    out_bufs = (out_v0, out_v1)
    isems = (isem0, isem1)
    osems = (osem0, osem1)

    pltpu.async_copy(table_hbm, table_v, tsem).wait()

    def idx_src(s):
        return idxt_hbm.at[pl.ds(s * _TS, _TS), pl.ds(b0, _BW)]

    def out_dst(s):
        return out_hbm.at[pl.ds(s * _TS * _D, _TS * _D), pl.ds(b0, _BW)]

    pltpu.async_copy(idx_src(0), idx_v0, isem0)
    pltpu.async_copy(idx_src(1), idx_v1, isem1)

    def outer(i, carry):
        for p in range(2):
            s = i * 2 + p
            idx_v = idx_bufs[p]
            out_v = out_bufs[p]
            pltpu.make_async_copy(idx_src(s), idx_v, isems[p]).wait()

            @pl.when(i > 0)
            def _():
                pltpu.make_async_copy(out_v, out_dst(s - 2), osems[p]).wait()

            @plsc.parallel_loop(0, _G, unroll=2)
            def group(g):
                c0 = g * _L
                for r in range(_TS):
                    v = idx_v[r, pl.ds(c0, _L)]
                    va = v * _D
                    for d in range(_D):
                        out_v[r * _D + d, pl.ds(c0, _L)] = plsc.load_gather(
                            table_v, [va + d]
                        )
            pltpu.async_copy(out_v, out_dst(s), osems[p])

            @pl.when(i < (_STEPS // 2 - 1))
            def _():
                pltpu.async_copy(idx_src(s + 2), idx_v, isems[p])

        return carry

    lax.fori_loop(0, _STEPS // 2, outer, 0)
    pltpu.make_async_copy(out_v0, out_dst(_STEPS - 2), osem0).wait()
    pltpu.make_async_copy(out_v1, out_dst(_STEPS - 1), osem1).wait()


def kernel(letter_idx, letter_embed):
    idxt = letter_idx.astype(jnp.int32).T
    table = letter_embed.reshape(_V * _D).astype(jnp.float32)

    mesh = plsc.VectorSubcoreMesh(core_axis_name="c", subcore_axis_name="s")
    k = pl.kernel(
        _body,
        mesh=mesh,
        compiler_params=pltpu.CompilerParams(needs_layout_passes=False),
        out_type=jax.ShapeDtypeStruct((_T * _D, _B), jnp.float32),
        scratch_types=[
            pltpu.VMEM((_V * _D,), jnp.float32),
            pltpu.VMEM((_TS, _BW), jnp.int32),
            pltpu.VMEM((_TS, _BW), jnp.int32),
            pltpu.VMEM((_TS * _D, _BW), jnp.float32),
            pltpu.VMEM((_TS * _D, _BW), jnp.float32),
            pltpu.SemaphoreType.DMA,
            pltpu.SemaphoreType.DMA,
            pltpu.SemaphoreType.DMA,
            pltpu.SemaphoreType.DMA,
            pltpu.SemaphoreType.DMA,
        ],
    )
    out2 = k(table, idxt)
    return out2.reshape(_T, _D, _B).transpose(2, 0, 1)
